# Initial kernel scaffold; baseline (speedup 1.0000x reference)
import functools

import jax
import jax.numpy as jnp
from jax import lax
from jax.experimental import pallas as pl
from jax.experimental.pallas import tpu as pltpu
from jax.experimental.pallas import tpu_sc as plsc

N = 10000
E = 320000
D = 128
R = 16
NB = 8

NC = 2
NS = 16
NW = NC * NS
EPT = E // NW
C = 80
K = EPT // C
RPS = N // NS


def _basis_body(wc_ref, wf_ref, o_ref):
    o_ref[...] = jnp.dot(wc_ref[...], wf_ref[...],
                         preferred_element_type=jnp.float32)


def _basis(w_comp, weight_flat):
    return pl.pallas_call(
        _basis_body,
        out_shape=jax.ShapeDtypeStruct((R, D * D), jnp.float32),
    )(w_comp, weight_flat)


TN = 400


def _transform_body(h_ref, wt_ref, rw_ref, t_ref, root_ref):
    h = h_ref[...]
    t_ref[...] = jnp.dot(h, wt_ref[...], preferred_element_type=jnp.float32)
    root_ref[...] = jnp.dot(h, rw_ref[...], preferred_element_type=jnp.float32)


def _transform(h, w_t, root_weight):
    return pl.pallas_call(
        _transform_body,
        grid=(N // TN,),
        in_specs=[
            pl.BlockSpec((TN, D), lambda i: (i, 0)),
            pl.BlockSpec((D, R * D), lambda i: (0, 0)),
            pl.BlockSpec((D, D), lambda i: (0, 0)),
        ],
        out_specs=[
            pl.BlockSpec((TN, R * D), lambda i: (i, 0)),
            pl.BlockSpec((TN, D), lambda i: (i, 0)),
        ],
        out_shape=[
            jax.ShapeDtypeStruct((N, R * D), jnp.float32),
            jax.ShapeDtypeStruct((N, D), jnp.float32),
        ],
    )(h, w_t, root_weight)


_sc_mesh = plsc.VectorSubcoreMesh(core_axis_name="c", subcore_axis_name="s")


@functools.partial(
    pl.kernel,
    out_type=jax.ShapeDtypeStruct((NC, N, D), jnp.float32),
    mesh=_sc_mesh,
    scratch_types=[
        pltpu.VMEM((K, C), jnp.int32),
        pltpu.VMEM((K, C), jnp.int32),
        pltpu.VMEM((C, D), jnp.float32),
        pltpu.VMEM_SHARED((N, D), jnp.float32),
        pltpu.SemaphoreType.DMA,
    ],
)
def _sc_edge_agg(table_hbm, key3_hbm, dst3_hbm, zrows_hbm, out_hbm,
                 keys2, dst2, rows, acc, sem):
    c = lax.axis_index("c")
    s = lax.axis_index("s")
    wid = s * NC + c
    pltpu.sync_copy(key3_hbm.at[wid], keys2)
    pltpu.sync_copy(dst3_hbm.at[wid], dst2)
    pltpu.sync_copy(zrows_hbm, acc.at[pl.ds(s * RPS, RPS)])
    plsc.subcore_barrier()

    def chunk(j, carry):
        pltpu.async_copy(table_hbm.at[keys2.at[j]], rows, sem).wait()
        pltpu.sync_copy(rows, acc.at[dst2.at[j]], add=True)
        return carry

    lax.fori_loop(0, K, chunk, 0)
    plsc.subcore_barrier()
    pltpu.sync_copy(acc.at[pl.ds(s * RPS, RPS)],
                    out_hbm.at[c, pl.ds(s * RPS, RPS)])


TF = 1000


def _final_body(root_ref, o_ref, bias_ref, out_ref):
    out_ref[...] = (root_ref[...] + o_ref[0] + o_ref[1] + bias_ref[...])


def _final(root, o, bias2d):
    return pl.pallas_call(
        _final_body,
        grid=(N // TF,),
        in_specs=[
            pl.BlockSpec((TF, D), lambda i: (i, 0)),
            pl.BlockSpec((NC, TF, D), lambda i: (0, i, 0)),
            pl.BlockSpec((1, D), lambda i: (0, 0)),
        ],
        out_specs=pl.BlockSpec((TF, D), lambda i: (i, 0)),
        out_shape=jax.ShapeDtypeStruct((N, D), jnp.float32),
    )(root, o, bias2d)


def kernel(h, edge_index, rel_type, weight, w_comp, root_weight, bias):
    src = edge_index[0].astype(jnp.int32)
    dst = edge_index[1].astype(jnp.int32)
    rel = rel_type.astype(jnp.int32)

    w = _basis(w_comp, weight.reshape(NB, D * D))
    w_t = w.reshape(R, D, D).transpose(1, 0, 2).reshape(D, R * D)

    t_flat, root = _transform(h, w_t, root_weight)
    table = t_flat.reshape(N * R, D)

    key3 = (src * R + rel).reshape(NW, K, C)
    dst3 = dst.reshape(NW, K, C)
    zrows = jnp.zeros((RPS, D), jnp.float32)

    o = _sc_edge_agg(table, key3, dst3, zrows)

    return _final(root, o, bias.reshape(1, D))


# speedup vs baseline: 3.0514x; 3.0514x over previous
import functools

import jax
import jax.numpy as jnp
from jax import lax
from jax.experimental import pallas as pl
from jax.experimental.pallas import tpu as pltpu
from jax.experimental.pallas import tpu_sc as plsc

N = 10000
E = 320000
D = 128
R = 16
NB = 8

NC = 2
NS = 16
NW = NC * NS
EPT = E // NW
C = 80
K = EPT // C
NPAD = 10240
RPS = NPAD // NS


def _basis_body(wc_ref, wf_ref, o_ref):
    o_ref[...] = jnp.dot(wc_ref[...], wf_ref[...],
                         preferred_element_type=jnp.float32)


def _basis(w_comp, weight_flat):
    return pl.pallas_call(
        _basis_body,
        out_shape=jax.ShapeDtypeStruct((R, D * D), jnp.float32),
    )(w_comp, weight_flat)


TN = 400


def _transform_body(h_ref, wt_ref, rw_ref, t_ref, root_ref):
    h = h_ref[...]
    t_ref[...] = jnp.dot(h, wt_ref[...], preferred_element_type=jnp.float32)
    root_ref[...] = jnp.dot(h, rw_ref[...], preferred_element_type=jnp.float32)


def _transform(h, w_t, root_weight):
    return pl.pallas_call(
        _transform_body,
        grid=(N // TN,),
        in_specs=[
            pl.BlockSpec((TN, D), lambda i: (i, 0)),
            pl.BlockSpec((D, R * D), lambda i: (0, 0)),
            pl.BlockSpec((D, D), lambda i: (0, 0)),
        ],
        out_specs=[
            pl.BlockSpec((TN, R * D), lambda i: (i, 0)),
            pl.BlockSpec((TN, D), lambda i: (i, 0)),
        ],
        out_shape=[
            jax.ShapeDtypeStruct((N, R * D), jnp.float32),
            jax.ShapeDtypeStruct((N, D), jnp.float32),
        ],
    )(h, w_t, root_weight)


def _sc_body(table_hbm, key3_hbm, dst3_hbm, zrows_hbm, out_hbm,
             keys2, dst2, rows, acc, sem):
    c = lax.axis_index("c")
    s = lax.axis_index("s")
    wid = s * NC + c
    pltpu.sync_copy(key3_hbm.at[wid], keys2)
    pltpu.sync_copy(dst3_hbm.at[wid], dst2)
    pltpu.sync_copy(zrows_hbm, acc.at[pl.ds(s * RPS, RPS)])
    plsc.subcore_barrier()

    def chunk(j, carry):
        pltpu.async_copy(table_hbm.at[keys2.at[j]], rows, sem).wait()
        pltpu.sync_copy(rows, acc.at[dst2.at[j]], add=True)
        return carry

    lax.fori_loop(0, K, chunk, 0)
    plsc.subcore_barrier()
    pltpu.sync_copy(acc.at[pl.ds(s * RPS, RPS)],
                    out_hbm.at[c, pl.ds(s * RPS, RPS)])


@functools.cache
def _sc_edge_agg():
    mesh = plsc.VectorSubcoreMesh(core_axis_name="c", subcore_axis_name="s",
                                  num_cores=NC, num_subcores=NS)
    return pl.kernel(
        _sc_body,
        out_type=jax.ShapeDtypeStruct((NC, NPAD, D), jnp.float32),
        mesh=mesh,
        scratch_types=[
            pltpu.VMEM((K, C), jnp.int32),
            pltpu.VMEM((K, C), jnp.int32),
            pltpu.VMEM((C, D), jnp.float32),
            pltpu.VMEM_SHARED((NPAD, D), jnp.float32),
            pltpu.SemaphoreType.DMA,
        ],
    )


TF = 1000


def _final_body(root_ref, o_ref, bias_ref, out_ref):
    out_ref[...] = (root_ref[...] + o_ref[0] + o_ref[1] + bias_ref[...])


def _final(root, o, bias2d):
    return pl.pallas_call(
        _final_body,
        grid=(N // TF,),
        in_specs=[
            pl.BlockSpec((TF, D), lambda i: (i, 0)),
            pl.BlockSpec((NC, TF, D), lambda i: (0, i, 0)),
            pl.BlockSpec((1, D), lambda i: (0, 0)),
        ],
        out_specs=pl.BlockSpec((TF, D), lambda i: (i, 0)),
        out_shape=jax.ShapeDtypeStruct((N, D), jnp.float32),
    )(root, o, bias2d)


def kernel(h, edge_index, rel_type, weight, w_comp, root_weight, bias):
    src = edge_index[0].astype(jnp.int32)
    dst = edge_index[1].astype(jnp.int32)
    rel = rel_type.astype(jnp.int32)

    w = _basis(w_comp, weight.reshape(NB, D * D))
    w_t = w.reshape(R, D, D).transpose(1, 0, 2).reshape(D, R * D)

    t_flat, root = _transform(h, w_t, root_weight)
    table = t_flat.reshape(N * R, D)

    key3 = (src * R + rel).reshape(NW, K, C)
    dst3 = dst.reshape(NW, K, C)
    zrows = jnp.zeros((RPS, D), jnp.float32)

    o = _sc_edge_agg()(table, key3, dst3, zrows)

    return _final(root, o, bias.reshape(1, D))


# speedup vs baseline: 3.9726x; 1.3019x over previous
import functools

import jax
import jax.numpy as jnp
from jax import lax
from jax.experimental import pallas as pl
from jax.experimental.pallas import tpu as pltpu
from jax.experimental.pallas import tpu_sc as plsc

N = 10000
E = 320000
D = 128
R = 16
NB = 8

NC = 2
NS = 16
NW = NC * NS
EPT = E // NW
C = 80
K = EPT // C
NPAD = 10240
RPS = NPAD // NS


def _basis_body(wc_ref, wf_ref, o_ref):
    o_ref[...] = jnp.dot(wc_ref[...], wf_ref[...],
                         preferred_element_type=jnp.float32)


def _basis(w_comp, weight_flat):
    return pl.pallas_call(
        _basis_body,
        out_shape=jax.ShapeDtypeStruct((R, D * D), jnp.float32),
    )(w_comp, weight_flat)


TN = 400


def _transform_body(h_ref, wt_ref, rw_ref, t_ref, root_ref):
    h = h_ref[...]
    t_ref[...] = jnp.dot(h, wt_ref[...], preferred_element_type=jnp.float32)
    root_ref[...] = jnp.dot(h, rw_ref[...], preferred_element_type=jnp.float32)


def _transform(h, w_t, root_weight):
    return pl.pallas_call(
        _transform_body,
        grid=(N // TN,),
        in_specs=[
            pl.BlockSpec((TN, D), lambda i: (i, 0)),
            pl.BlockSpec((D, R * D), lambda i: (0, 0)),
            pl.BlockSpec((D, D), lambda i: (0, 0)),
        ],
        out_specs=[
            pl.BlockSpec((TN, R * D), lambda i: (i, 0)),
            pl.BlockSpec((TN, D), lambda i: (i, 0)),
        ],
        out_shape=[
            jax.ShapeDtypeStruct((N, R * D), jnp.float32),
            jax.ShapeDtypeStruct((N, D), jnp.float32),
        ],
    )(h, w_t, root_weight)


def _sc_body(table_hbm, key2_hbm, dst3_hbm, zrows_hbm, out_hbm,
             keys1, dst2, rows0, rows1, acc, semg0, semg1):
    c = lax.axis_index("c")
    s = lax.axis_index("s")
    wid = s * NC + c
    pltpu.sync_copy(key2_hbm.at[wid], keys1)
    pltpu.sync_copy(dst3_hbm.at[wid], dst2)
    pltpu.sync_copy(zrows_hbm, acc.at[pl.ds(s * RPS, RPS)])
    plsc.subcore_barrier()

    pltpu.async_copy(table_hbm.at[keys1.at[pl.ds(0, C)]], rows0, semg0)

    def pair(p, carry):
        j0 = 2 * p
        pltpu.async_copy(table_hbm.at[keys1.at[pl.ds((j0 + 1) * C, C)]], rows1, semg1)
        pltpu.make_async_copy(table_hbm.at[keys1.at[pl.ds(j0 * C, C)]], rows0, semg0).wait()
        pltpu.sync_copy(rows0, acc.at[dst2.at[j0]], add=True)
        pltpu.async_copy(table_hbm.at[keys1.at[pl.ds((j0 + 2) * C, C)]], rows0, semg0)
        pltpu.make_async_copy(table_hbm.at[keys1.at[pl.ds((j0 + 1) * C, C)]], rows1,
                              semg1).wait()
        pltpu.sync_copy(rows1, acc.at[dst2.at[j0 + 1]], add=True)
        return carry

    lax.fori_loop(0, (K - 1) // 2, pair, 0)
    pltpu.make_async_copy(table_hbm.at[keys1.at[pl.ds((K - 1) * C, C)]], rows0, semg0).wait()
    pltpu.sync_copy(rows0, acc.at[dst2.at[K - 1]], add=True)
    plsc.subcore_barrier()
    pltpu.sync_copy(acc.at[pl.ds(s * RPS, RPS)],
                    out_hbm.at[c, pl.ds(s * RPS, RPS)])


@functools.cache
def _sc_edge_agg():
    mesh = plsc.VectorSubcoreMesh(core_axis_name="c", subcore_axis_name="s",
                                  num_cores=NC, num_subcores=NS)
    return pl.kernel(
        _sc_body,
        out_type=jax.ShapeDtypeStruct((NC, NPAD, D), jnp.float32),
        mesh=mesh,
        scratch_types=[
            pltpu.VMEM((EPT,), jnp.int32),
            pltpu.VMEM((K, C), jnp.int32),
            pltpu.VMEM((C, D), jnp.float32),
            pltpu.VMEM((C, D), jnp.float32),
            pltpu.VMEM_SHARED((NPAD, D), jnp.float32),
            pltpu.SemaphoreType.DMA,
            pltpu.SemaphoreType.DMA,
        ],
    )


TF = 1000


def _final_body(root_ref, o_ref, bias_ref, out_ref):
    out_ref[...] = (root_ref[...] + o_ref[0] + o_ref[1] + bias_ref[...])


def _final(root, o, bias2d):
    return pl.pallas_call(
        _final_body,
        grid=(N // TF,),
        in_specs=[
            pl.BlockSpec((TF, D), lambda i: (i, 0)),
            pl.BlockSpec((NC, TF, D), lambda i: (0, i, 0)),
            pl.BlockSpec((1, D), lambda i: (0, 0)),
        ],
        out_specs=pl.BlockSpec((TF, D), lambda i: (i, 0)),
        out_shape=jax.ShapeDtypeStruct((N, D), jnp.float32),
    )(root, o, bias2d)


def kernel(h, edge_index, rel_type, weight, w_comp, root_weight, bias):
    src = edge_index[0].astype(jnp.int32)
    dst = edge_index[1].astype(jnp.int32)
    rel = rel_type.astype(jnp.int32)

    w = _basis(w_comp, weight.reshape(NB, D * D))
    w_t = w.reshape(R, D, D).transpose(1, 0, 2).reshape(D, R * D)

    t_flat, root = _transform(h, w_t, root_weight)
    table = t_flat.reshape(N * R, D)

    key2 = (src * R + rel).reshape(NW, EPT)
    dst3 = dst.reshape(NW, K, C)
    zrows = jnp.zeros((RPS, D), jnp.float32)

    o = _sc_edge_agg()(table, key2, dst3, zrows)

    return _final(root, o, bias.reshape(1, D))
